# Initial kernel scaffold; baseline (speedup 1.0000x reference)
import jax
import jax.numpy as jnp
from jax.experimental import pallas as pl


def kernel(x, target):
    raise NotImplementedError("write your pallas kernel here")


# speedup vs baseline: 6.3053x; 6.3053x over previous
import math

import jax
import jax.numpy as jnp
from jax.experimental import pallas as pl
from jax.experimental.pallas import tpu as pltpu

_SIZE = 32000
_ROWS = 4096
_SMOOTH = 0.1
_CONF = 1.0 - _SMOOTH
_EPS = _SMOOTH / (_SIZE - 2)
_C_CONST = (_SIZE - 2) * _EPS * math.log(_EPS) + _CONF * math.log(_CONF)

_RB = 512
_CB = 3200


def _reduce_body(tgt_ref, x_ref, out_ref):
    i = pl.program_id(0)
    j = pl.program_id(1)

    @pl.when((i == 0) & (j == 0))
    def _init():
        out_ref[...] = jnp.zeros_like(out_ref)

    tgt = tgt_ref[...]
    valid = (tgt != 0).astype(jnp.float32)
    xb = x_ref[...]

    acc = -_EPS * jnp.sum(xb * valid)

    col0 = jnp.sum(xb[:, 0:1] * valid)
    nvalid = jnp.sum(valid)
    acc = acc + jnp.where(j == 0, _EPS * col0 + _C_CONST * nvalid, 0.0)

    col_ids = jax.lax.broadcasted_iota(jnp.int32, (_RB, _CB), 1) + j * _CB
    match = jnp.where(col_ids == tgt, valid, 0.0)
    acc = acc + (_EPS - _CONF) * jnp.sum(xb * match)

    out_ref[...] += acc


@jax.jit
def kernel(x, target):
    tgt = target.astype(jnp.int32).reshape(_ROWS, 1)
    out = pl.pallas_call(
        _reduce_body,
        grid=(_ROWS // _RB, _SIZE // _CB),
        in_specs=[
            pl.BlockSpec((_RB, 1), lambda i, j: (i, 0)),
            pl.BlockSpec((_RB, _CB), lambda i, j: (i, j)),
        ],
        out_specs=pl.BlockSpec((1, 1), lambda i, j: (0, 0)),
        out_shape=jax.ShapeDtypeStruct((1, 1), jnp.float32),
    )(tgt, x)
    return out[0, 0]
